# baseline (speedup 1.0000x reference)
import functools

import jax
import jax.numpy as jnp
from jax import lax
from jax.experimental import pallas as pl
from jax.experimental.pallas import tpu as pltpu
from jax.experimental.pallas import tpu_sc as plsc

_N = 10000
_D = 128
_WALK_LENGTH = 5
_NUM_WALKS = 10
_WINDOW = 2
_NEG = 5
_LAMBDA = 1.0

_NC = 2
_NS = 16
_TILES = _NC * _NS

_B = 128
_CHUNK = 24
_NSLOT = 3
_NHALF = 2
_PAIRS_PER_CHUNK = _B * _CHUNK

_UNIQ = [(i, j) for i in range(_WALK_LENGTH)
         for j in range(i + 1, min(_WALK_LENGTH, i + _WINDOW + 1))]
_SRC_SEQ = [i for i in range(_WALK_LENGTH)
            for j in range(max(0, i - _WINDOW), min(_WALK_LENGTH, i + _WINDOW + 1))
            if j != i]

_NWALK = _N * _NUM_WALKS
_NPAIR = _NWALK * len(_SRC_SEQ)
_NPOS = _NWALK * len(_UNIQ)
_NNEG = _NPAIR * _NEG

_POS_CHUNKS = -(-_NPOS // (_TILES * _PAIRS_PER_CHUNK))
_NEG_CHUNKS = -(-_NNEG // (_TILES * _PAIRS_PER_CHUNK))
_NPOS_PAD = _TILES * _POS_CHUNKS * _PAIRS_PER_CHUNK
_NNEG_PAD = _TILES * _NEG_CHUNKS * _PAIRS_PER_CHUNK

_LN2 = 0.6931471805599453


def _build_csr_idx(edge_index, num_nodes):
    src = edge_index[0]
    dst = edge_index[1]
    order = jnp.argsort(src)
    dst_sorted = dst[order]
    deg = jnp.bincount(src, length=num_nodes)
    offsets = jnp.concatenate([jnp.zeros((1,), dtype=deg.dtype), jnp.cumsum(deg)])
    return dst_sorted, deg, offsets


def _gen_walks(edge_index, num_nodes, key):
    dst_sorted, deg, offsets = _build_csr_idx(edge_index, num_nodes)
    starts = jnp.tile(jnp.arange(num_nodes, dtype=jnp.int32), _NUM_WALKS)
    cur = starts
    walk_cols = [cur]
    for step in range(_WALK_LENGTH - 1):
        k = jax.random.fold_in(key, step)
        u = jax.random.uniform(k, cur.shape)
        d = deg[cur]
        r = jnp.floor(u * jnp.maximum(d, 1).astype(jnp.float32)).astype(jnp.int32)
        r = jnp.minimum(r, jnp.maximum(d - 1, 0))
        nxt = dst_sorted[offsets[cur] + r]
        nxt = jnp.where(d > 0, nxt, cur)
        walk_cols.append(nxt)
        cur = nxt
    return jnp.stack(walk_cols, axis=1)


def _softplus16(x):
    ax = jnp.abs(x)
    y = 1.0 + jnp.exp(-ax)
    bits = plsc.bitcast(y, jnp.int32)
    eexp = jnp.right_shift(bits, 23) - 127
    m = plsc.bitcast(
        jnp.bitwise_or(jnp.bitwise_and(bits, 0x007FFFFF), 0x3F800000),
        jnp.float32)
    t = (m - 1.0) / (m + 1.0)
    t2 = t * t
    lnm = t * (2.0 + t2 * (0.66666667 + t2 * (0.4 + t2 * 0.28571429)))
    return jnp.maximum(x, 0.0) + _LN2 * eexp.astype(jnp.float32) + lnm


def _sc_loss_call(table, pos_a, pos_b, neg_a, neg_b):
    mesh = plsc.VectorSubcoreMesh(
        core_axis_name="c", subcore_axis_name="s",
        num_cores=_NC, num_subcores=_NS)

    @functools.partial(
        pl.kernel,
        out_type=jax.ShapeDtypeStruct((_TILES, 32), jnp.float32),
        mesh=mesh,
        compiler_params=pltpu.CompilerParams(needs_layout_passes=False),
        scratch_types=[
            pltpu.VMEM((_CHUNK, _B), jnp.int32),
            pltpu.VMEM((_CHUNK, _B), jnp.int32),
            pltpu.VMEM((_NSLOT, _B, _D), jnp.float32),
            pltpu.VMEM((_NSLOT, _B, _D), jnp.float32),
            pltpu.VMEM((_B,), jnp.float32),
            pltpu.VMEM((32,), jnp.float32),
        ] + [pltpu.SemaphoreType.DMA] * (2 * _NSLOT * _NHALF),
    )
    def sc_loss(table_h, pos_a_h, pos_b_h, neg_a_h, neg_b_h, out_h,
                idx_a, idx_b, rows_a, rows_b, dotbuf, ostage, *sems):
        wid = lax.axis_index("s") * _NC + lax.axis_index("c")
        _BH = _B // _NHALF

        def _sem(slot, side, h):
            return sems[(slot * 2 + side) * _NHALF + h]

        def fire(b, slot):
            for side, (idx, rows) in enumerate(((idx_a, rows_a),
                                                (idx_b, rows_b))):
                for h in range(_NHALF):
                    pltpu.async_copy(
                        table_h.at[idx.at[b, pl.ds(h * _BH, _BH)]],
                        rows.at[slot, pl.ds(h * _BH, _BH)],
                        _sem(slot, side, h))

        def drain(b, slot):
            for side, (idx, rows) in enumerate(((idx_a, rows_a),
                                                (idx_b, rows_b))):
                for h in range(_NHALF):
                    pltpu.make_async_copy(
                        table_h.at[idx.at[b, pl.ds(h * _BH, _BH)]],
                        rows.at[slot, pl.ds(h * _BH, _BH)],
                        _sem(slot, side, h)).wait()

        lane = lax.iota(jnp.int32, 16)
        last_mask = lane == 15

        def batch_compute(acc, slot, negate):
            def pair_body(p):
                dotv = (rows_a[slot, p, pl.ds(0, 16)]
                        * rows_b[slot, p, pl.ds(0, 16)])
                for c in range(1, _D // 16):
                    dotv = dotv + (rows_a[slot, p, pl.ds(c * 16, 16)]
                                   * rows_b[slot, p, pl.ds(c * 16, 16)])
                csum = plsc.cumsum(dotv)
                plsc.store_scatter(dotbuf, [jnp.zeros((16,), jnp.int32) + p],
                                   csum, mask=last_mask)

            pl.loop(0, _B, unroll=4)(pair_body)

            def group_body(g, acc):
                x = dotbuf[pl.ds(g * 16, 16)]
                x = -x if negate else x
                return acc + _softplus16(x)

            return pl.loop(0, _B // 16, init_carry=acc)(group_body)

        def run_phase(pa_h, pb_h, nchunks, negate):
            rows_per_tile = nchunks * _CHUNK

            def chunk_body(ci, acc):
                base = wid * rows_per_tile + ci * _CHUNK
                pltpu.sync_copy(pa_h.at[pl.ds(base, _CHUNK)], idx_a)
                pltpu.sync_copy(pb_h.at[pl.ds(base, _CHUNK)], idx_b)
                for j in range(_NSLOT - 1):
                    fire(j, j)

                def ring_body(b, acc):
                    for j in range(_NSLOT):
                        bb = b + j

                        @pl.when(bb + _NSLOT - 1 < _CHUNK)
                        def _():
                            fire(bb + _NSLOT - 1, (j + _NSLOT - 1) % _NSLOT)

                        drain(bb, j)
                        acc = batch_compute(acc, j, negate)
                    return acc

                return pl.loop(0, _CHUNK, step=_NSLOT, init_carry=acc)(ring_body)

            return pl.loop(0, nchunks,
                           init_carry=jnp.zeros((16,), jnp.float32))(chunk_body)

        acc_pos = run_phase(pos_a_h, pos_b_h, _POS_CHUNKS, True)
        acc_neg = run_phase(neg_a_h, neg_b_h, _NEG_CHUNKS, False)
        ostage[pl.ds(0, 16)] = acc_pos
        ostage[pl.ds(16, 16)] = acc_neg
        pltpu.sync_copy(ostage, out_h.at[wid])

    return sc_loss(table, pos_a, pos_b, neg_a, neg_b)


def _pad2d(idx, total):
    pad = total - idx.shape[0]
    idx = jnp.concatenate([idx, jnp.full((pad,), _N, jnp.int32)])
    return idx.reshape(-1, _B)


def kernel(embeddings, edge_index):
    num_nodes = embeddings.shape[0]
    key = jax.random.key(42)
    walks = _gen_walks(edge_index, num_nodes, key)

    src_nodes = jnp.concatenate([walks[:, i] for i in _SRC_SEQ])
    pos_a = jnp.concatenate([walks[:, i] for i, _ in _UNIQ])
    pos_b = jnp.concatenate([walks[:, j] for _, j in _UNIQ])
    neg_b = jnp.concatenate([
        jax.random.randint(jax.random.fold_in(key, 1000 + s), src_nodes.shape,
                           0, num_nodes, dtype=jnp.int32)
        for s in range(_NEG)])
    neg_a = jnp.tile(src_nodes, _NEG)

    table = jnp.concatenate(
        [embeddings, jnp.zeros((8, _D), embeddings.dtype)], axis=0)

    sums = _sc_loss_call(
        table,
        _pad2d(pos_a, _NPOS_PAD), _pad2d(pos_b, _NPOS_PAD),
        _pad2d(neg_a, _NNEG_PAD), _pad2d(neg_b, _NNEG_PAD))

    s_pos = jnp.sum(sums[:, :16]) - (_NPOS_PAD - _NPOS) * _LN2
    s_neg = jnp.sum(sums[:, 16:]) - (_NNEG_PAD - _NNEG) * _LN2
    pos_term = 2.0 * s_pos / _NPAIR
    neg_term = s_neg / _NNEG
    return _LAMBDA * (pos_term + neg_term)


# speedup vs baseline: 1.6328x; 1.6328x over previous
import functools

import jax
import jax.numpy as jnp
from jax import lax
from jax.experimental import pallas as pl
from jax.experimental.pallas import tpu as pltpu
from jax.experimental.pallas import tpu_sc as plsc

_N = 10000
_D = 128
_WALK_LENGTH = 5
_NUM_WALKS = 10
_WINDOW = 2
_NEG = 5
_LAMBDA = 1.0

_NC = 2
_NS = 16
_TILES = _NC * _NS

_B = 192
_CHUNK = 8
_NSLOT = 2
_PAIRS_PER_CHUNK = _B * _CHUNK

_UNIQ = [(i, j) for i in range(_WALK_LENGTH)
         for j in range(i + 1, min(_WALK_LENGTH, i + _WINDOW + 1))]
_SRC_SEQ = [i for i in range(_WALK_LENGTH)
            for j in range(max(0, i - _WINDOW), min(_WALK_LENGTH, i + _WINDOW + 1))
            if j != i]

_NWALK = _N * _NUM_WALKS
_NPAIR = _NWALK * len(_SRC_SEQ)
_NPOS = _NWALK * len(_UNIQ)
_NNEG = _NPAIR * _NEG

_POS_CHUNKS = -(-_NPOS // (_TILES * _PAIRS_PER_CHUNK))
_NEG_CHUNKS = -(-_NNEG // (_TILES * _PAIRS_PER_CHUNK))
_NPOS_PAD = _TILES * _POS_CHUNKS * _PAIRS_PER_CHUNK
_NNEG_PAD = _TILES * _NEG_CHUNKS * _PAIRS_PER_CHUNK

_LN2 = 0.6931471805599453


def _build_csr_idx(edge_index, num_nodes):
    src = edge_index[0]
    dst = edge_index[1]
    order = jnp.argsort(src)
    dst_sorted = dst[order]
    deg = jnp.bincount(src, length=num_nodes)
    offsets = jnp.concatenate([jnp.zeros((1,), dtype=deg.dtype), jnp.cumsum(deg)])
    return dst_sorted, deg, offsets


def _gen_walks(edge_index, num_nodes, key):
    dst_sorted, deg, offsets = _build_csr_idx(edge_index, num_nodes)
    starts = jnp.tile(jnp.arange(num_nodes, dtype=jnp.int32), _NUM_WALKS)
    cur = starts
    walk_cols = [cur]
    for step in range(_WALK_LENGTH - 1):
        k = jax.random.fold_in(key, step)
        u = jax.random.uniform(k, cur.shape)
        d = deg[cur]
        r = jnp.floor(u * jnp.maximum(d, 1).astype(jnp.float32)).astype(jnp.int32)
        r = jnp.minimum(r, jnp.maximum(d - 1, 0))
        nxt = dst_sorted[offsets[cur] + r]
        nxt = jnp.where(d > 0, nxt, cur)
        walk_cols.append(nxt)
        cur = nxt
    return jnp.stack(walk_cols, axis=1)


def _softplus16(x):
    ax = jnp.abs(x)
    y = 1.0 + jnp.exp(-ax)
    bits = plsc.bitcast(y, jnp.int32)
    eexp = jnp.right_shift(bits, 23) - 127
    m = plsc.bitcast(
        jnp.bitwise_or(jnp.bitwise_and(bits, 0x007FFFFF), 0x3F800000),
        jnp.float32)
    t = (m - 1.0) / (m + 1.0)
    t2 = t * t
    lnm = t * (2.0 + t2 * (0.66666667 + t2 * (0.4 + t2 * 0.28571429)))
    return jnp.maximum(x, 0.0) + _LN2 * eexp.astype(jnp.float32) + lnm


def _sc_loss_call(table, pos_ab, neg_ab):
    mesh = plsc.VectorSubcoreMesh(
        core_axis_name="c", subcore_axis_name="s",
        num_cores=_NC, num_subcores=_NS)

    @functools.partial(
        pl.kernel,
        out_type=jax.ShapeDtypeStruct((_TILES, 32), jnp.float32),
        mesh=mesh,
        compiler_params=pltpu.CompilerParams(needs_layout_passes=False),
        scratch_types=[
            pltpu.VMEM((_CHUNK * 2 * _B,), jnp.int32),
            pltpu.VMEM((_NSLOT, 2 * _B, _D), jnp.float32),
            pltpu.VMEM((_B,), jnp.float32),
            pltpu.VMEM((32,), jnp.float32),
        ] + [pltpu.SemaphoreType.DMA] * _NSLOT,
    )
    def sc_loss(table_h, pos_h, neg_h, out_h,
                idx_ab, rows, dotbuf, ostage, *sems):
        wid = lax.axis_index("s") * _NC + lax.axis_index("c")

        def _idx_at(b):
            off = pl.multiple_of(b * (2 * _B), 2 * _B)
            return idx_ab.at[pl.ds(off, 2 * _B)]

        def fire(b, slot):
            pltpu.async_copy(table_h.at[_idx_at(b)], rows.at[slot],
                             sems[slot])

        def drain(b, slot):
            pltpu.make_async_copy(table_h.at[_idx_at(b)], rows.at[slot],
                                  sems[slot]).wait()

        lane = lax.iota(jnp.int32, 16)
        last_mask = lane == 15

        def batch_compute(acc, slot, negate):
            def pair_body(p):
                dotv = (rows[slot, p, pl.ds(0, 16)]
                        * rows[slot, _B + p, pl.ds(0, 16)])
                for c in range(1, _D // 16):
                    dotv = dotv + (rows[slot, p, pl.ds(c * 16, 16)]
                                   * rows[slot, _B + p, pl.ds(c * 16, 16)])
                csum = plsc.cumsum(dotv)
                plsc.store_scatter(dotbuf, [jnp.zeros((16,), jnp.int32) + p],
                                   csum, mask=last_mask)

            pl.loop(0, _B, unroll=4)(pair_body)

            def group_body(g, acc):
                x = dotbuf[pl.ds(g * 16, 16)]
                x = -x if negate else x
                return acc + _softplus16(x)

            return pl.loop(0, _B // 16, init_carry=acc)(group_body)

        def run_phase(pab_h, nchunks, negate):
            rows_per_tile = nchunks * _CHUNK

            def chunk_body(ci, acc):
                base = wid * rows_per_tile + ci * _CHUNK
                off = pl.multiple_of(base * (2 * _B), _CHUNK * 2 * _B)
                pltpu.sync_copy(pab_h.at[pl.ds(off, _CHUNK * 2 * _B)], idx_ab)
                for j in range(_NSLOT - 1):
                    fire(j, j)

                def ring_body(b, acc):
                    for j in range(_NSLOT):
                        bb = b + j

                        @pl.when(bb + _NSLOT - 1 < _CHUNK)
                        def _():
                            fire(bb + _NSLOT - 1, (j + _NSLOT - 1) % _NSLOT)

                        drain(bb, j)
                        acc = batch_compute(acc, j, negate)
                    return acc

                return pl.loop(0, _CHUNK, step=_NSLOT, init_carry=acc)(ring_body)

            return pl.loop(0, nchunks,
                           init_carry=jnp.zeros((16,), jnp.float32))(chunk_body)

        acc_pos = run_phase(pos_h, _POS_CHUNKS, True)
        acc_neg = run_phase(neg_h, _NEG_CHUNKS, False)
        ostage[pl.ds(0, 16)] = acc_pos
        ostage[pl.ds(16, 16)] = acc_neg
        pltpu.sync_copy(ostage, out_h.at[wid])

    return sc_loss(table, pos_ab, neg_ab)


def _merge_ab(a, b, total):
    pad = total - a.shape[0]
    fill = jnp.full((pad,), _N, jnp.int32)
    a2 = jnp.concatenate([a, fill]).reshape(-1, _B)
    b2 = jnp.concatenate([b, fill]).reshape(-1, _B)
    return jnp.concatenate([a2, b2], axis=1).reshape(-1)


def kernel(embeddings, edge_index):
    num_nodes = embeddings.shape[0]
    key = jax.random.key(42)
    walks = _gen_walks(edge_index, num_nodes, key)

    src_nodes = jnp.concatenate([walks[:, i] for i in _SRC_SEQ])
    pos_a = jnp.concatenate([walks[:, i] for i, _ in _UNIQ])
    pos_b = jnp.concatenate([walks[:, j] for _, j in _UNIQ])
    neg_b = jnp.concatenate([
        jax.random.randint(jax.random.fold_in(key, 1000 + s), src_nodes.shape,
                           0, num_nodes, dtype=jnp.int32)
        for s in range(_NEG)])
    neg_a = jnp.tile(src_nodes, _NEG)

    table = jnp.concatenate(
        [embeddings, jnp.zeros((8, _D), embeddings.dtype)], axis=0)

    sums = _sc_loss_call(
        table,
        _merge_ab(pos_a, pos_b, _NPOS_PAD),
        _merge_ab(neg_a, neg_b, _NNEG_PAD))

    s_pos = jnp.sum(sums[:, :16]) - (_NPOS_PAD - _NPOS) * _LN2
    s_neg = jnp.sum(sums[:, 16:]) - (_NNEG_PAD - _NNEG) * _LN2
    pos_term = 2.0 * s_pos / _NPAIR
    neg_term = s_neg / _NNEG
    return _LAMBDA * (pos_term + neg_term)


# speedup vs baseline: 1.8736x; 1.1475x over previous
import functools

import jax
import jax.numpy as jnp
from jax import lax
from jax.experimental import pallas as pl
from jax.experimental.pallas import tpu as pltpu
from jax.experimental.pallas import tpu_sc as plsc

_N = 10000
_D = 128
_WALK_LENGTH = 5
_NUM_WALKS = 10
_WINDOW = 2
_NEG = 5
_LAMBDA = 1.0

_NC = 2
_NS = 16
_TILES = _NC * _NS

_B = 128
_CHUNK = 16
_PAIRS_PER_CHUNK = _B * _CHUNK

_UNIQ = [(i, j) for i in range(_WALK_LENGTH)
         for j in range(i + 1, min(_WALK_LENGTH, i + _WINDOW + 1))]
_SRC_SEQ = [i for i in range(_WALK_LENGTH)
            for j in range(max(0, i - _WINDOW), min(_WALK_LENGTH, i + _WINDOW + 1))
            if j != i]

_NWALK = _N * _NUM_WALKS
_NPAIR = _NWALK * len(_SRC_SEQ)
_NPOS = _NWALK * len(_UNIQ)
_NNEG = _NPAIR * _NEG

_POS_CHUNKS = -(-_NPOS // (_TILES * _PAIRS_PER_CHUNK))
_NEG_CHUNKS = -(-_NNEG // (_TILES * _PAIRS_PER_CHUNK))
_NPOS_PAD = _TILES * _POS_CHUNKS * _PAIRS_PER_CHUNK
_NNEG_PAD = _TILES * _NEG_CHUNKS * _PAIRS_PER_CHUNK

_LN2 = 0.6931471805599453


def _build_csr_idx(edge_index, num_nodes):
    src = edge_index[0]
    dst = edge_index[1]
    order = jnp.argsort(src)
    dst_sorted = dst[order]
    deg = jnp.bincount(src, length=num_nodes)
    offsets = jnp.concatenate([jnp.zeros((1,), dtype=deg.dtype), jnp.cumsum(deg)])
    return dst_sorted, deg, offsets


def _gen_walks(edge_index, num_nodes, key):
    dst_sorted, deg, offsets = _build_csr_idx(edge_index, num_nodes)
    starts = jnp.tile(jnp.arange(num_nodes, dtype=jnp.int32), _NUM_WALKS)
    cur = starts
    walk_cols = [cur]
    for step in range(_WALK_LENGTH - 1):
        k = jax.random.fold_in(key, step)
        u = jax.random.uniform(k, cur.shape)
        d = deg[cur]
        r = jnp.floor(u * jnp.maximum(d, 1).astype(jnp.float32)).astype(jnp.int32)
        r = jnp.minimum(r, jnp.maximum(d - 1, 0))
        nxt = dst_sorted[offsets[cur] + r]
        nxt = jnp.where(d > 0, nxt, cur)
        walk_cols.append(nxt)
        cur = nxt
    return jnp.stack(walk_cols, axis=1)


def _softplus16(x):
    ax = jnp.abs(x)
    y = 1.0 + jnp.exp(-ax)
    bits = plsc.bitcast(y, jnp.int32)
    eexp = jnp.right_shift(bits, 23) - 127
    m = plsc.bitcast(
        jnp.bitwise_or(jnp.bitwise_and(bits, 0x007FFFFF), 0x3F800000),
        jnp.float32)
    t = (m - 1.0) / (m + 1.0)
    t2 = t * t
    lnm = t * (2.0 + t2 * (0.66666667 + t2 * (0.4 + t2 * 0.28571429)))
    return jnp.maximum(x, 0.0) + _LN2 * eexp.astype(jnp.float32) + lnm


def _sc_loss_call(table, pos_a, pos_b, neg_a, neg_b):
    mesh = plsc.VectorSubcoreMesh(
        core_axis_name="c", subcore_axis_name="s",
        num_cores=_NC, num_subcores=_NS)

    @functools.partial(
        pl.kernel,
        out_type=jax.ShapeDtypeStruct((_TILES, 32), jnp.float32),
        mesh=mesh,
        compiler_params=pltpu.CompilerParams(needs_layout_passes=False),
        scratch_types=[
            pltpu.VMEM((_CHUNK, _B), jnp.int32),
            pltpu.VMEM((_CHUNK, _B), jnp.int32),
            pltpu.VMEM((2, _B, _D), jnp.float32),
            pltpu.VMEM((2, _B, _D), jnp.float32),
            pltpu.VMEM((_B,), jnp.float32),
            pltpu.VMEM((32,), jnp.float32),
            pltpu.SemaphoreType.DMA,
            pltpu.SemaphoreType.DMA,
            pltpu.SemaphoreType.DMA,
            pltpu.SemaphoreType.DMA,
        ],
    )
    def sc_loss(table_h, pos_a_h, pos_b_h, neg_a_h, neg_b_h, out_h,
                idx_a, idx_b, rows_a, rows_b, dotbuf, ostage,
                sem_a0, sem_a1, sem_b0, sem_b1):
        wid = lax.axis_index("s") * _NC + lax.axis_index("c")
        sems_a = (sem_a0, sem_a1)
        sems_b = (sem_b0, sem_b1)

        def fire(b, slot):
            pltpu.async_copy(table_h.at[idx_a.at[b]], rows_a.at[slot],
                             sems_a[slot])
            pltpu.async_copy(table_h.at[idx_b.at[b]], rows_b.at[slot],
                             sems_b[slot])

        def drain(b, slot):
            pltpu.make_async_copy(table_h.at[idx_a.at[b]], rows_a.at[slot],
                                  sems_a[slot]).wait()
            pltpu.make_async_copy(table_h.at[idx_b.at[b]], rows_b.at[slot],
                                  sems_b[slot]).wait()

        lane = lax.iota(jnp.int32, 16)
        last_mask = lane == 15

        def batch_compute(acc, slot, negate):
            def pair_body(p):
                dotv = (rows_a[slot, p, pl.ds(0, 16)]
                        * rows_b[slot, p, pl.ds(0, 16)])
                for c in range(1, _D // 16):
                    dotv = dotv + (rows_a[slot, p, pl.ds(c * 16, 16)]
                                   * rows_b[slot, p, pl.ds(c * 16, 16)])
                csum = plsc.cumsum(dotv)
                plsc.store_scatter(dotbuf, [jnp.zeros((16,), jnp.int32) + p],
                                   csum, mask=last_mask)

            pl.loop(0, _B, unroll=2)(pair_body)

            def group_body(g, acc):
                x = dotbuf[pl.ds(g * 16, 16)]
                x = -x if negate else x
                return acc + _softplus16(x)

            return pl.loop(0, _B // 16, init_carry=acc)(group_body)

        def run_phase(pa_h, pb_h, nchunks, negate):
            rows_per_tile = nchunks * _CHUNK

            def chunk_body(ci, acc):
                base = wid * rows_per_tile + ci * _CHUNK
                pltpu.sync_copy(pa_h.at[pl.ds(base, _CHUNK)], idx_a)
                pltpu.sync_copy(pb_h.at[pl.ds(base, _CHUNK)], idx_b)
                fire(0, 0)

                def two_body(b, acc):
                    fire(b + 1, 1)
                    drain(b, 0)
                    acc = batch_compute(acc, 0, negate)

                    @pl.when(b + 2 < _CHUNK)
                    def _():
                        fire(b + 2, 0)

                    drain(b + 1, 1)
                    return batch_compute(acc, 1, negate)

                return pl.loop(0, _CHUNK, step=2, init_carry=acc)(two_body)

            return pl.loop(0, nchunks,
                           init_carry=jnp.zeros((16,), jnp.float32))(chunk_body)

        acc_pos = run_phase(pos_a_h, pos_b_h, _POS_CHUNKS, True)
        acc_neg = run_phase(neg_a_h, neg_b_h, _NEG_CHUNKS, False)
        ostage[pl.ds(0, 16)] = acc_pos
        ostage[pl.ds(16, 16)] = acc_neg
        pltpu.sync_copy(ostage, out_h.at[wid])

    return sc_loss(table, pos_a, pos_b, neg_a, neg_b)


def _pad2d(idx, total):
    pad = total - idx.shape[0]
    idx = jnp.concatenate([idx, jnp.full((pad,), _N, jnp.int32)])
    return idx.reshape(-1, _B)


def kernel(embeddings, edge_index):
    num_nodes = embeddings.shape[0]
    key = jax.random.key(42)
    walks = _gen_walks(edge_index, num_nodes, key)

    src_nodes = jnp.concatenate([walks[:, i] for i in _SRC_SEQ])
    pos_a = jnp.concatenate([walks[:, i] for i, _ in _UNIQ])
    pos_b = jnp.concatenate([walks[:, j] for _, j in _UNIQ])
    neg_b = jnp.concatenate([
        jax.random.randint(jax.random.fold_in(key, 1000 + s), src_nodes.shape,
                           0, num_nodes, dtype=jnp.int32)
        for s in range(_NEG)])
    neg_a = jnp.tile(src_nodes, _NEG)

    table = jnp.concatenate(
        [embeddings, jnp.zeros((8, _D), embeddings.dtype)], axis=0)

    sums = _sc_loss_call(
        table,
        _pad2d(pos_a, _NPOS_PAD), _pad2d(pos_b, _NPOS_PAD),
        _pad2d(neg_a, _NNEG_PAD), _pad2d(neg_b, _NNEG_PAD))

    s_pos = jnp.sum(sums[:, :16]) - (_NPOS_PAD - _NPOS) * _LN2
    s_neg = jnp.sum(sums[:, 16:]) - (_NNEG_PAD - _NNEG) * _LN2
    pos_term = 2.0 * s_pos / _NPAIR
    neg_term = s_neg / _NNEG
    return _LAMBDA * (pos_term + neg_term)
